# baseline (device time: 593335 ns/iter reference)
import jax
import jax.numpy as jnp
from jax import lax
from jax.experimental import pallas as pl
from jax.experimental.pallas import tpu as pltpu

N_DEV = 4
M = 8192
D = 2048
C = M // N_DEV
H = C // 2
LANES = 2
GENS = 2
R = H // (LANES * GENS)
NSTEP = 2 * (N_DEV - 1)
NG = GENS * NSTEP


def kernel(partial, resid, gamma):
    partial2d = partial.reshape(M, D)
    gamma2d = gamma.reshape(1, D)

    def body(partial_ref, resid_ref, gamma_ref, out_ref,
             commA, commB, stageA, stageB, rstageA, rstageB,
             send_semsA, recv_semsA, send_semsB, recv_semsB,
             local_semsA, local_semsB, out_semsA, out_semsB,
             rsemsA, rsemsB,
             creditA0, creditA1, creditB0, creditB1):
        my = lax.axis_index("i")
        right = lax.rem(my + 1, N_DEV)
        left = lax.rem(my + N_DEV - 1, N_DEV)

        credits = {("A", 0): creditA0, ("A", 1): creditA1,
                   ("B", 0): creditB0, ("B", 1): creditB1}
        comms = {"A": commA, "B": commB}
        stages = {"A": stageA, "B": stageB}
        rstages = {"A": rstageA, "B": rstageB}
        send_sems = {"A": send_semsA, "B": send_semsB}
        recv_sems = {"A": recv_semsA, "B": recv_semsB}
        local_sems = {"A": local_semsA, "B": local_semsB}
        out_sems = {"A": out_semsA, "B": out_semsB}
        rsems = {"A": rsemsA, "B": rsemsB}
        halfs = {"A": 0, "B": 1}
        send_to = {"A": right, "B": left}
        credit_to = {"A": left, "B": right}
        owned = {"A": lax.rem(my + 1, N_DEV),
                 "B": lax.rem(my + N_DEV - 1, N_DEV)}

        barrier_sem = pltpu.get_barrier_semaphore()
        for nbr in (left, right):
            pl.semaphore_signal(
                barrier_sem, inc=1,
                device_id=(nbr,), device_id_type=pl.DeviceIdType.MESH,
            )
        pl.semaphore_wait(barrier_sem, 2)

        def rowoff(idx, d, lane, v):
            return idx * C + halfs[d] * H + (lane * GENS + v) * R

        def load(src_ref, idx, d, lane, v, dst, sem=None):
            cp = pltpu.make_async_copy(
                src_ref.at[pl.ds(rowoff(idx, d, lane, v), R), :],
                dst, local_sems[d].at[lane] if sem is None else sem)
            cp.start()
            return cp

        def store_out(d, lane, slot, idx, v):
            cp = pltpu.make_async_copy(
                comms[d].at[lane, slot],
                out_ref.at[pl.ds(rowoff(idx, d, lane, v), R), :],
                out_sems[d].at[lane])
            cp.start()
            return cp

        inflight = {}
        staged = {}
        rpend = {}
        pend = {(d, l): None for d in "AB" for l in range(LANES)}

        def issue(lane, G):
            t = G % NSTEP
            v = G // NSTEP
            ss = G % 2
            rs = (G + 1) % 2
            if t == 0 and v == 0:
                cps = [load(partial_ref, my, d, lane, v,
                            comms[d].at[lane, ss]) for d in "AB"]
                for cp in cps:
                    cp.wait()
            if t == 0 and v > 0:
                for d in "AB":
                    rpend[(d, lane)].wait()
            if G >= 1:
                for d in "AB":
                    pl.semaphore_wait(credits[(d, lane)], 1)
            for d in "AB":
                src = (rstages[d].at[lane] if (t == 0 and v > 0)
                       else comms[d].at[lane, ss])
                rdma = pltpu.make_async_remote_copy(
                    src_ref=src,
                    dst_ref=comms[d].at[lane, rs],
                    send_sem=send_sems[d].at[lane, ss],
                    recv_sem=recv_sems[d].at[lane, rs],
                    device_id=(send_to[d],),
                    device_id_type=pl.DeviceIdType.MESH,
                )
                rdma.start()
                inflight[(d, lane)] = rdma
            if t < N_DEV - 1:
                idx = {"A": lax.rem(my + N_DEV - 1 - t, N_DEV),
                       "B": lax.rem(my + t + 1, N_DEV)}
                for d in "AB":
                    staged[(d, lane)] = load(
                        partial_ref, idx[d], d, lane, v, stages[d].at[lane])
            if t == 1:
                for d in "AB":
                    rpend[(d, lane)] = load(
                        resid_ref, owned[d], d, lane, v,
                        rstages[d].at[lane], rsems[d].at[lane])
            if t == N_DEV - 1 and v < GENS - 1:
                for d in "AB":
                    rpend[(d, lane)] = load(
                        partial_ref, my, d, lane, v + 1,
                        rstages[d].at[lane], rsems[d].at[lane])

        def complete(lane, G):
            t = G % NSTEP
            v = G // NSTEP
            rs = (G + 1) % 2
            for d in "AB":
                pend_cp = pend[(d, lane)]
                if pend_cp is not None:
                    pend_cp.wait()
                    pend[(d, lane)] = None
                inflight[(d, lane)].wait_send()
            if G < NG - 1:
                for d in "AB":
                    pl.semaphore_signal(
                        credits[(d, lane)], inc=1,
                        device_id=(credit_to[d],),
                        device_id_type=pl.DeviceIdType.MESH)
            for d in "AB":
                inflight[(d, lane)].wait_recv()
            if t < N_DEV - 1:
                for d in "AB":
                    staged[(d, lane)].wait()
                    comm, stage = comms[d], stages[d]
                    comm[lane, rs, :, :] = (
                        comm[lane, rs, :, :] + stage[lane, :, :])
                if t == N_DEV - 2:
                    for d in "AB":
                        rpend[(d, lane)].wait()
                    for d in "AB":
                        comm, rstage = comms[d], rstages[d]
                        y = comm[lane, rs, :, :] + rstage[lane, :, :]
                        inv = lax.rsqrt(
                            jnp.mean(y * y, axis=-1, keepdims=True)
                            + 1e-6)
                        comm[lane, rs, :, :] = y * inv * gamma_ref[:, :]
                        pend[(d, lane)] = store_out(
                            d, lane, rs, owned[d], v)
            else:
                h = t - (N_DEV - 1)
                idx = {"A": lax.rem(my + N_DEV - h, N_DEV),
                       "B": lax.rem(my + h, N_DEV)}
                for d in "AB":
                    pend[(d, lane)] = store_out(d, lane, rs, idx[d], v)

        issue(0, 0)
        issue(1, 0)
        for G in range(NG):
            for lane in range(LANES):
                complete(lane, G)
                if G < NG - 1:
                    issue(lane, G + 1)
        for d in "AB":
            for lane in range(LANES):
                pend[(d, lane)].wait()

    return pl.pallas_call(
        body,
        out_shape=jax.ShapeDtypeStruct((M, D), jnp.float32),
        in_specs=[
            pl.BlockSpec(memory_space=pl.ANY),
            pl.BlockSpec(memory_space=pl.ANY),
            pl.BlockSpec(memory_space=pltpu.VMEM),
        ],
        out_specs=pl.BlockSpec(memory_space=pl.ANY),
        scratch_shapes=[
            pltpu.VMEM((LANES, 2, R, D), jnp.float32),
            pltpu.VMEM((LANES, 2, R, D), jnp.float32),
            pltpu.VMEM((LANES, R, D), jnp.float32),
            pltpu.VMEM((LANES, R, D), jnp.float32),
            pltpu.VMEM((LANES, R, D), jnp.float32),
            pltpu.VMEM((LANES, R, D), jnp.float32),
            pltpu.SemaphoreType.DMA((LANES, 2)),
            pltpu.SemaphoreType.DMA((LANES, 2)),
            pltpu.SemaphoreType.DMA((LANES, 2)),
            pltpu.SemaphoreType.DMA((LANES, 2)),
            pltpu.SemaphoreType.DMA((LANES,)),
            pltpu.SemaphoreType.DMA((LANES,)),
            pltpu.SemaphoreType.DMA((LANES,)),
            pltpu.SemaphoreType.DMA((LANES,)),
            pltpu.SemaphoreType.DMA((LANES,)),
            pltpu.SemaphoreType.DMA((LANES,)),
            pltpu.SemaphoreType.REGULAR,
            pltpu.SemaphoreType.REGULAR,
            pltpu.SemaphoreType.REGULAR,
            pltpu.SemaphoreType.REGULAR,
        ],
        compiler_params=pltpu.CompilerParams(
            collective_id=0,
            vmem_limit_bytes=60 * 1024 * 1024,
        ),
    )(partial2d, resid, gamma2d)
